# baseline (speedup 1.0000x reference)
import functools

import jax
import jax.numpy as jnp
from jax import lax
from jax.experimental import pallas as pl
from jax.experimental.pallas import tpu as pltpu
from jax.experimental.pallas import tpu_sc as plsc

B = 16384
NW = 32
BPW = B // NW
C = 64
NCH = BPW // C
L = 16

ITEM_ROWS = 1000000
PACKED_BLK = 2048
TR_GRID = (ITEM_ROWS // 2 + PACKED_BLK - 1) // PACKED_BLK
H_SPLIT = TR_GRID * PACKED_BLK


def _transpose_body(xa_ref, xb_ref, o_ref):
    x = jnp.concatenate([xa_ref[...], xb_ref[...]], axis=0)
    eye = jnp.eye(128, dtype=jnp.float32)
    dn = (((0,), (0,)), ((), ()))
    o_ref[...] = lax.dot_general(x, eye, dn,
                                 preferred_element_type=jnp.float32)


def _pack_item(item_t):
    return pl.pallas_call(
        _transpose_body,
        grid=(TR_GRID,),
        in_specs=[
            pl.BlockSpec((64, PACKED_BLK), lambda c: (0, c)),
            pl.BlockSpec(
                (64, PACKED_BLK),
                lambda c: (0, jnp.minimum(c + TR_GRID,
                                          (ITEM_ROWS - 1) // PACKED_BLK)),
            ),
        ],
        out_specs=pl.BlockSpec((PACKED_BLK, 128), lambda c: (c, 0)),
        out_shape=jax.ShapeDtypeStruct((H_SPLIT, 128), jnp.float32),
        compiler_params=pltpu.CompilerParams(
            fuse_transposed_lhs_in_matmul=True),
    )(item_t, item_t)


def _sc_body(u_idx, pi_idx, hp_idx, pc_idx, pb_idx, ni_idx, hn_idx, nc_idx,
             nb_idx,
             user_table, item_pk, cat_table, brand_table,
             out_hbm,
             idx_v, u_rows, pi_rows, pc_rows, pb_rows,
             ni_rows, nc_rows, nb_rows, score_v,
             sem_u, sem_i, sem_pc, sem_pb, sem_nc, sem_nb):
    nc_ax = jax.lax.axis_index("c")
    ns_ax = jax.lax.axis_index("s")
    wid = ns_ax * 2 + nc_ax
    base = wid * BPW

    def chunk_body(c, _):
        pltpu.sync_copy(u_idx.at[wid, c], idx_v.at[0])
        pltpu.sync_copy(pi_idx.at[wid, c], idx_v.at[1])
        pltpu.sync_copy(hp_idx.at[wid, c], idx_v.at[2])
        pltpu.sync_copy(pc_idx.at[wid, c], idx_v.at[3])
        pltpu.sync_copy(pb_idx.at[wid, c], idx_v.at[4])
        pltpu.sync_copy(ni_idx.at[wid, c], idx_v.at[5])
        pltpu.sync_copy(hn_idx.at[wid, c], idx_v.at[6])
        pltpu.sync_copy(nc_idx.at[wid, c], idx_v.at[7])
        pltpu.sync_copy(nb_idx.at[wid, c], idx_v.at[8])

        cp_u = pltpu.async_copy(user_table.at[idx_v.at[0]], u_rows, sem_u)
        cp_pi = pltpu.async_copy(item_pk.at[idx_v.at[1]], pi_rows, sem_i)
        cp_ni = pltpu.async_copy(item_pk.at[idx_v.at[5]], ni_rows, sem_i)

        def fire_group(g, _):
            b16 = g * L
            d16 = pl.ds(b16, L)
            vc_p = idx_v[3, d16]
            vb_p = idx_v[4, d16]
            vc_n = idx_v[7, d16]
            vb_n = idx_v[8, d16]
            for j in range(L):
                r = b16 + j
                pltpu.async_copy(cat_table.at[vc_p[j]], pc_rows.at[r], sem_pc)
                pltpu.async_copy(brand_table.at[vb_p[j]], pb_rows.at[r], sem_pb)
                pltpu.async_copy(cat_table.at[vc_n[j]], nc_rows.at[r], sem_nc)
                pltpu.async_copy(brand_table.at[vb_n[j]], nb_rows.at[r], sem_nb)
            return 0

        lax.fori_loop(0, C // L, fire_group, 0)

        d32 = cat_table.at[pl.ds(0, C)]
        pltpu.make_async_copy(d32, pc_rows, sem_pc).wait()
        pltpu.make_async_copy(d32, pb_rows, sem_pb).wait()
        pltpu.make_async_copy(d32, nc_rows, sem_nc).wait()
        pltpu.make_async_copy(d32, nb_rows, sem_nb).wait()
        cp_u.wait()
        cp_pi.wait()
        cp_ni.wait()

        def group_body(g, _):
            rowv = lax.iota(jnp.int32, L) + g * L
            d16 = pl.ds(g * L, L)
            hpv = idx_v[2, d16]
            hnv = idx_v[6, d16]
            acc = jnp.zeros((L,), jnp.float32)
            for d in range(64):
                dv = jnp.full((L,), d, jnp.int32)
                uv = plsc.load_gather(u_rows, [rowv, dv])
                pv = plsc.load_gather(pi_rows, [rowv, hpv + d])
                nv = plsc.load_gather(ni_rows, [rowv, hnv + d])
                acc += uv * (nv - pv)
            for d in range(32):
                dv = jnp.full((L,), d, jnp.int32)
                uv = plsc.load_gather(u_rows, [rowv, dv + 64])
                pv = plsc.load_gather(pc_rows, [rowv, dv])
                nv = plsc.load_gather(nc_rows, [rowv, dv])
                acc += uv * (nv - pv)
            for d in range(32):
                dv = jnp.full((L,), d, jnp.int32)
                uv = plsc.load_gather(u_rows, [rowv, dv + 96])
                pv = plsc.load_gather(pb_rows, [rowv, dv])
                nv = plsc.load_gather(nb_rows, [rowv, dv])
                acc += uv * (nv - pv)
            score_v[pl.ds(g * L, L)] = acc
            return 0

        lax.fori_loop(0, C // L, group_body, 0)
        pltpu.sync_copy(score_v, out_hbm.at[pl.ds(base + c * C, C)])
        return 0

    lax.fori_loop(0, NCH, chunk_body, 0)


@jax.jit
def _sc_scores(u_idx, pi_idx, hp_idx, pc_idx, pb_idx, ni_idx, hn_idx, nc_idx,
               nb_idx, user_table, item_pk, cat_table, brand_table):
    mesh = plsc.VectorSubcoreMesh(core_axis_name="c", subcore_axis_name="s")
    f = functools.partial(
        pl.kernel,
        mesh=mesh,
        compiler_params=pltpu.CompilerParams(needs_layout_passes=False),
        out_type=jax.ShapeDtypeStruct((B,), jnp.float32),
        scratch_types=[
            pltpu.VMEM((9, C), jnp.int32),
            pltpu.VMEM((C, 128), jnp.float32),
            pltpu.VMEM((C, 128), jnp.float32),
            pltpu.VMEM((C, 32), jnp.float32),
            pltpu.VMEM((C, 32), jnp.float32),
            pltpu.VMEM((C, 128), jnp.float32),
            pltpu.VMEM((C, 32), jnp.float32),
            pltpu.VMEM((C, 32), jnp.float32),
            pltpu.VMEM((C,), jnp.float32),
            pltpu.SemaphoreType.DMA,
            pltpu.SemaphoreType.DMA,
            pltpu.SemaphoreType.DMA,
            pltpu.SemaphoreType.DMA,
            pltpu.SemaphoreType.DMA,
            pltpu.SemaphoreType.DMA,
        ],
    )(_sc_body)
    return f(u_idx, pi_idx, hp_idx, pc_idx, pb_idx, ni_idx, hn_idx, nc_idx,
             nb_idx, user_table, item_pk, cat_table, brand_table)


def _tc_loss_body(s_ref, o_ref):
    s = s_ref[...]
    sp = jnp.maximum(s, 0.0) + jnp.log1p(jnp.exp(-jnp.abs(s)))
    o_ref[...] = (jnp.sum(sp) * (1.0 / B)).reshape(1, 1)


def _tc_loss(scores):
    out = pl.pallas_call(
        _tc_loss_body,
        out_shape=jax.ShapeDtypeStruct((1, 1), jnp.float32),
    )(scores.reshape(128, 128))
    return out[0, 0]


def kernel(user, item, item_cat, item_brand, neg_item, neg_item_cat,
           neg_item_brand, user_table, item_table, cat_table, brand_table):
    def rs(x):
        return x.astype(jnp.int32).reshape(NW, NCH, C)

    item32 = item.astype(jnp.int32)
    neg32 = neg_item.astype(jnp.int32)
    item_pk = _pack_item(item_table.T)
    ip_row = jnp.where(item32 < H_SPLIT, item32, item32 - H_SPLIT)
    hp = jnp.where(item32 < H_SPLIT, 0, 64).astype(jnp.int32)
    in_row = jnp.where(neg32 < H_SPLIT, neg32, neg32 - H_SPLIT)
    hn = jnp.where(neg32 < H_SPLIT, 0, 64).astype(jnp.int32)
    scores = _sc_scores(
        rs(user), rs(ip_row), rs(hp), rs(item_cat),
        rs(item_brand), rs(in_row), rs(hn),
        rs(neg_item_cat), rs(neg_item_brand),
        user_table, item_pk, cat_table, brand_table)
    return _tc_loss(scores)


# speedup vs baseline: 1.1751x; 1.1751x over previous
import functools

import jax
import jax.numpy as jnp
from jax import lax
from jax.experimental import pallas as pl
from jax.experimental.pallas import tpu as pltpu
from jax.experimental.pallas import tpu_sc as plsc

B = 16384
NW = 32
BPW = B // NW
C = 64
NCH = BPW // C
L = 16

ITEM_ROWS = 1000000
PACKED_BLK = 2048
TR_GRID = (ITEM_ROWS // 2 + PACKED_BLK - 1) // PACKED_BLK
H_SPLIT = TR_GRID * PACKED_BLK


def _transpose_body(xa_ref, xb_ref, o_ref):
    x = jnp.concatenate([xa_ref[...], xb_ref[...]], axis=0)
    eye = jnp.eye(128, dtype=jnp.float32)
    dn = (((0,), (0,)), ((), ()))
    o_ref[...] = lax.dot_general(x, eye, dn,
                                 preferred_element_type=jnp.float32)


def _pack_item(item_t):
    return pl.pallas_call(
        _transpose_body,
        grid=(TR_GRID,),
        in_specs=[
            pl.BlockSpec((64, PACKED_BLK), lambda c: (0, c)),
            pl.BlockSpec(
                (64, PACKED_BLK),
                lambda c: (0, jnp.minimum(c + TR_GRID,
                                          (ITEM_ROWS - 1) // PACKED_BLK)),
            ),
        ],
        out_specs=pl.BlockSpec((PACKED_BLK, 128), lambda c: (c, 0)),
        out_shape=jax.ShapeDtypeStruct((H_SPLIT, 128), jnp.float32),
        compiler_params=pltpu.CompilerParams(
            fuse_transposed_lhs_in_matmul=True),
    )(item_t, item_t)


def _sc_body(u_idx, pi_idx, hp_idx, pc_idx, pb_idx, ni_idx, hn_idx, nc_idx,
             nb_idx,
             user_table, item_pk, cat_table, brand_table,
             out_hbm,
             idx_v, u_rows, pi_rows, pc_rows, pb_rows,
             ni_rows, nc_rows, nb_rows, partial,
             sem_u, sem_i, sem_pc, sem_pb, sem_nc, sem_nb):
    nc_ax = jax.lax.axis_index("c")
    ns_ax = jax.lax.axis_index("s")
    wid = ns_ax * 2 + nc_ax
    base = wid * BPW

    def chunk_body(c, _):
        pltpu.sync_copy(u_idx.at[wid, c], idx_v.at[0])
        pltpu.sync_copy(pi_idx.at[wid, c], idx_v.at[1])
        pltpu.sync_copy(hp_idx.at[wid, c], idx_v.at[2])
        pltpu.sync_copy(pc_idx.at[wid, c], idx_v.at[3])
        pltpu.sync_copy(pb_idx.at[wid, c], idx_v.at[4])
        pltpu.sync_copy(ni_idx.at[wid, c], idx_v.at[5])
        pltpu.sync_copy(hn_idx.at[wid, c], idx_v.at[6])
        pltpu.sync_copy(nc_idx.at[wid, c], idx_v.at[7])
        pltpu.sync_copy(nb_idx.at[wid, c], idx_v.at[8])

        cp_u = pltpu.async_copy(user_table.at[idx_v.at[0]], u_rows, sem_u)
        cp_pi = pltpu.async_copy(item_pk.at[idx_v.at[1]], pi_rows, sem_i)
        cp_ni = pltpu.async_copy(item_pk.at[idx_v.at[5]], ni_rows, sem_i)

        def fire_group(g, _):
            b16 = g * L
            d16 = pl.ds(b16, L)
            vc_p = idx_v[3, d16]
            vb_p = idx_v[4, d16]
            vc_n = idx_v[7, d16]
            vb_n = idx_v[8, d16]
            for j in range(L):
                r = b16 + j
                pltpu.async_copy(cat_table.at[vc_p[j]], pc_rows.at[r], sem_pc)
                pltpu.async_copy(brand_table.at[vb_p[j]], pb_rows.at[r], sem_pb)
                pltpu.async_copy(cat_table.at[vc_n[j]], nc_rows.at[r], sem_nc)
                pltpu.async_copy(brand_table.at[vb_n[j]], nb_rows.at[r], sem_nb)
            return 0

        lax.fori_loop(0, C // L, fire_group, 0)

        d32 = cat_table.at[pl.ds(0, C)]
        pltpu.make_async_copy(d32, pc_rows, sem_pc).wait()
        pltpu.make_async_copy(d32, pb_rows, sem_pb).wait()
        pltpu.make_async_copy(d32, nc_rows, sem_nc).wait()
        pltpu.make_async_copy(d32, nb_rows, sem_nb).wait()
        cp_u.wait()
        cp_pi.wait()
        cp_ni.wait()

        def group_body(g, _):
            b16 = g * L
            d16 = pl.ds(b16, L)
            hpv = idx_v[2, d16]
            hnv = idx_v[6, d16]
            for j in range(L):
                r = b16 + j
                hp_j = hpv[j]
                hn_j = hnv[j]
                acc = jnp.zeros((L,), jnp.float32)
                for k in range(4):
                    du = pl.ds(k * L, L)
                    pv = pi_rows[r, pl.ds(hp_j + k * L, L)]
                    nv = ni_rows[r, pl.ds(hn_j + k * L, L)]
                    acc += u_rows[r, du] * (nv - pv)
                for k in range(2):
                    d = pl.ds(k * L, L)
                    du = pl.ds(64 + k * L, L)
                    acc += u_rows[r, du] * (nc_rows[r, d] - pc_rows[r, d])
                for k in range(2):
                    d = pl.ds(k * L, L)
                    du = pl.ds(96 + k * L, L)
                    acc += u_rows[r, du] * (nb_rows[r, d] - pb_rows[r, d])
                partial[r, :] = acc
            return 0

        lax.fori_loop(0, C // L, group_body, 0)
        pltpu.sync_copy(partial, out_hbm.at[pl.ds(base + c * C, C)])
        return 0

    lax.fori_loop(0, NCH, chunk_body, 0)


@jax.jit
def _sc_scores(u_idx, pi_idx, hp_idx, pc_idx, pb_idx, ni_idx, hn_idx, nc_idx,
               nb_idx, user_table, item_pk, cat_table, brand_table):
    mesh = plsc.VectorSubcoreMesh(core_axis_name="c", subcore_axis_name="s")
    f = functools.partial(
        pl.kernel,
        mesh=mesh,
        compiler_params=pltpu.CompilerParams(needs_layout_passes=False),
        out_type=jax.ShapeDtypeStruct((B, L), jnp.float32),
        scratch_types=[
            pltpu.VMEM((9, C), jnp.int32),
            pltpu.VMEM((C, 128), jnp.float32),
            pltpu.VMEM((C, 128), jnp.float32),
            pltpu.VMEM((C, 32), jnp.float32),
            pltpu.VMEM((C, 32), jnp.float32),
            pltpu.VMEM((C, 128), jnp.float32),
            pltpu.VMEM((C, 32), jnp.float32),
            pltpu.VMEM((C, 32), jnp.float32),
            pltpu.VMEM((C, L), jnp.float32),
            pltpu.SemaphoreType.DMA,
            pltpu.SemaphoreType.DMA,
            pltpu.SemaphoreType.DMA,
            pltpu.SemaphoreType.DMA,
            pltpu.SemaphoreType.DMA,
            pltpu.SemaphoreType.DMA,
        ],
    )(_sc_body)
    return f(u_idx, pi_idx, hp_idx, pc_idx, pb_idx, ni_idx, hn_idx, nc_idx,
             nb_idx, user_table, item_pk, cat_table, brand_table)


def _tc_loss_body(p_ref, o_ref):
    s = jnp.sum(p_ref[...], axis=1, keepdims=True)
    sp = jnp.maximum(s, 0.0) + jnp.log1p(jnp.exp(-jnp.abs(s)))
    o_ref[...] = (jnp.sum(sp) * (1.0 / B)).reshape(1, 1)


def _tc_loss(partials):
    out = pl.pallas_call(
        _tc_loss_body,
        out_shape=jax.ShapeDtypeStruct((1, 1), jnp.float32),
    )(partials)
    return out[0, 0]


def kernel(user, item, item_cat, item_brand, neg_item, neg_item_cat,
           neg_item_brand, user_table, item_table, cat_table, brand_table):
    def rs(x):
        return x.astype(jnp.int32).reshape(NW, NCH, C)

    item32 = item.astype(jnp.int32)
    neg32 = neg_item.astype(jnp.int32)
    item_pk = _pack_item(item_table.T)
    ip_row = jnp.where(item32 < H_SPLIT, item32, item32 - H_SPLIT)
    hp = jnp.where(item32 < H_SPLIT, 0, 64).astype(jnp.int32)
    in_row = jnp.where(neg32 < H_SPLIT, neg32, neg32 - H_SPLIT)
    hn = jnp.where(neg32 < H_SPLIT, 0, 64).astype(jnp.int32)
    scores = _sc_scores(
        rs(user), rs(ip_row), rs(hp), rs(item_cat),
        rs(item_brand), rs(in_row), rs(hn),
        rs(neg_item_cat), rs(neg_item_brand),
        user_table, item_pk, cat_table, brand_table)
    return _tc_loss(scores)
